# baseline (speedup 1.0000x reference)
import functools

import jax
import jax.numpy as jnp
from jax import lax
from jax.experimental import pallas as pl
from jax.experimental.pallas import tpu as pltpu
from jax.experimental.pallas import tpu_sc as plsc

WS = 8
KNN = 9
EPS_BN = 1e-5
NPW = WS * WS
GW = 8
GSZ = GW * NPW

_F32 = jnp.float32
_I32 = jnp.int32
_HI = lax.Precision.HIGHEST
_RBN = float(1.0 / (1.0 + EPS_BN) ** 0.5)


def _dot(a, b, dims, precision=_HI):
    return lax.dot_general(a, b, (dims, ((), ())),
                           preferred_element_type=_F32, precision=precision)


def _tc1_body(xw_ref, w1_ref, b1_ref, g1_ref, be1_ref, wa_ref, wb_ref,
              bg_ref, af_ref, bft_ref, idx_ref):
    xb = xw_ref[...]

    y = _dot(w1_ref[...], xb, ((1,), (0,)), precision=None)
    y = (y + b1_ref[...]) * _RBN * g1_ref[...] + be1_ref[...]

    ss = jnp.sum(y * y, axis=0, keepdims=True)
    inv = 1.0 / jnp.maximum(jnp.sqrt(ss), 1e-12)
    xn = y * inv
    sq = jnp.sum(xn * xn, axis=0, keepdims=True)

    af = _dot(wa_ref[...], y, ((1,), (0,)), precision=None) + bg_ref[...]
    bf = _dot(wb_ref[...], y, ((1,), (0,)))
    af_ref[...] = af
    bfp = jnp.concatenate([bf, jnp.zeros((1, GSZ), _F32)], axis=0)
    bft_ref[...] = jnp.transpose(bfp)

    dts = []
    for g in range(GW):
        sl = slice(g * NPW, (g + 1) * NPW)
        p = xn[:, sl]
        gm = _dot(p, p, ((0,), (0,)), precision=None)
        sqg = sq[:, sl]
        dts.append((sqg + (-2.0 * gm)) + jnp.transpose(sqg))
    dwt = jnp.concatenate(dts, axis=1)

    rowid = lax.broadcasted_iota(_I32, (NPW, GSZ), 0)
    firsts = []
    for _ in range(KNN):
        cmin = jnp.min(dwt, axis=0, keepdims=True)
        first = jnp.min(jnp.where(dwt == cmin, rowid, NPW),
                        axis=0, keepdims=True)
        firsts.append(first)
        onehot = rowid == first
        dwt = jnp.where(onehot, jnp.inf, dwt)
    firsts += [firsts[0]] * (16 - KNN)
    idx_ref[...] = jnp.concatenate(firsts, axis=0)


def _tc2_body(xw_ref, af_ref, mt_ref, w2_ref, b2_ref, out_ref):
    c2 = af_ref.shape[0]
    m = jnp.transpose(mt_ref[:, :c2])
    e = jnp.maximum(af_ref[...] + m, 0.0)
    out = _dot(w2_ref[...], e, ((1,), (0,)), precision=None) \
        + b2_ref[...] + xw_ref[...]
    out_ref[...] = out


def _sc_gather_max(bft, idx16, tot, c2):
    info = plsc.get_sparse_core_info()
    nw = info.num_cores * info.num_subcores
    wins_per_tile = (tot // NPW) // nw
    st = c2 + 1
    blk = NPW * st
    mesh = plsc.VectorSubcoreMesh(core_axis_name="c", subcore_axis_name="s")

    @functools.partial(
        pl.kernel, mesh=mesh,
        out_type=jax.ShapeDtypeStruct((tot * st,), _F32),
        compiler_params=pltpu.CompilerParams(use_tc_tiling_on_sc=False,
                                             needs_layout_passes=False),
        scratch_types=[
            pltpu.VMEM((2, blk), _F32),
            pltpu.VMEM((2, 16, NPW), _I32),
            pltpu.VMEM((2, blk), _F32),
            pltpu.SemaphoreType.DMA((2,)),
            pltpu.SemaphoreType.DMA((2,)),
        ],
    )
    def sck(bft_hbm, idx_hbm, m_hbm, bft_v, idx_v, m_v, rsem, wsem):
        wid = lax.axis_index("s") * info.num_cores + lax.axis_index("c")

        def start_read(w, sl):
            widx = wid * wins_per_tile + w
            base = widx * NPW
            h1 = pltpu.async_copy(bft_hbm.at[pl.ds(base * st, blk)],
                                  bft_v.at[sl], rsem.at[sl])
            h2 = pltpu.async_copy(idx_hbm.at[widx], idx_v.at[sl],
                                  rsem.at[sl])
            return h1, h2

        hs = {0: start_read(0, 0)}
        ws = {}
        for w in range(wins_per_tile):
            sl = w % 2
            if w + 1 < wins_per_tile:
                hs[w + 1] = start_read(w + 1, (w + 1) % 2)
            for h in hs.pop(w):
                h.wait()
            if w >= 2:
                ws.pop(w - 2).wait()
            for ng in range(NPW // 16):
                jvecs = [idx_v[sl, k, pl.ds(ng * 16, 16)] * st
                         for k in range(KNN)]
                nvec = (lax.broadcasted_iota(_I32, (16,), 0) + ng * 16) * st

                @plsc.parallel_loop(0, c2, unroll=4)
                def chan(c):
                    g = [plsc.load_gather(bft_v.at[sl], [jvecs[k] + c])
                         for k in range(KNN)]
                    while len(g) > 1:
                        g = [jnp.maximum(g[i], g[i + 1])
                             for i in range(0, len(g) - 1, 2)] \
                            + ([g[-1]] if len(g) % 2 else [])
                    plsc.store_scatter(m_v.at[sl], [nvec + c], g[0])
            base = (wid * wins_per_tile + w) * NPW
            ws[w] = pltpu.async_copy(m_v.at[sl],
                                     m_hbm.at[pl.ds(base * st, blk)],
                                     wsem.at[sl])
        for h in ws.values():
            h.wait()

    idx3 = idx16.reshape(16, tot // NPW, NPW).transpose(1, 0, 2)
    return sck(bft.reshape(-1), idx3)


def kernel(x, fc1_w, fc1_b, bn1_g, bn1_b, gc_w, gc_b, gc_bn_g, gc_bn_b,
           fc2_w, fc2_b, bn2_g, bn2_b):
    b, c, h, w = x.shape
    nwh, nww = h // WS, w // WS
    tot = b * nwh * nww * NPW
    c2 = 2 * c

    r = 1.0 / jnp.sqrt(jnp.float32(1.0 + EPS_BN))
    sg = gc_bn_g * r
    wg = gc_w * sg[:, None]
    bgv = gc_b * sg + gc_bn_b
    wa = wg[:, :c] - wg[:, c:]
    wb = wg[:, c:]
    s2 = bn2_g * r
    w2 = fc2_w * s2[:, None]
    b2 = fc2_b * s2 + bn2_b

    xw = x.reshape(b, c, nwh, WS, nww, WS)
    xw = jnp.transpose(xw, (1, 0, 2, 4, 3, 5)).reshape(c, tot)

    def tc1(xw_h, tot_h):
        return pl.pallas_call(
            _tc1_body,
            grid=(tot_h // GSZ,),
            in_specs=[
                pl.BlockSpec((c, GSZ), lambda i: (0, i)),
                pl.BlockSpec((c, c), lambda i: (0, 0)),
                pl.BlockSpec((c, 1), lambda i: (0, 0)),
                pl.BlockSpec((c, 1), lambda i: (0, 0)),
                pl.BlockSpec((c, 1), lambda i: (0, 0)),
                pl.BlockSpec((c2, c), lambda i: (0, 0)),
                pl.BlockSpec((c2, c), lambda i: (0, 0)),
                pl.BlockSpec((c2, 1), lambda i: (0, 0)),
            ],
            out_specs=[
                pl.BlockSpec((c2, GSZ), lambda i: (0, i)),
                pl.BlockSpec((GSZ, c2 + 1), lambda i: (i, 0)),
                pl.BlockSpec((16, GSZ), lambda i: (0, i)),
            ],
            out_shape=[
                jax.ShapeDtypeStruct((c2, tot_h), _F32),
                jax.ShapeDtypeStruct((tot_h, c2 + 1), _F32),
                jax.ShapeDtypeStruct((16, tot_h), _I32),
            ],
        )(xw_h, fc1_w, fc1_b[:, None], bn1_g[:, None], bn1_b[:, None],
          wa, wb, bgv[:, None])

    def tc2(xw_h, af_h, mt_h, tot_h):
        return pl.pallas_call(
            _tc2_body,
            grid=(tot_h // GSZ,),
            in_specs=[
                pl.BlockSpec((c, GSZ), lambda i: (0, i)),
                pl.BlockSpec((c2, GSZ), lambda i: (0, i)),
                pl.BlockSpec((GSZ, c2 + 1), lambda i: (i, 0)),
                pl.BlockSpec((c, c2), lambda i: (0, 0)),
                pl.BlockSpec((c, 1), lambda i: (0, 0)),
            ],
            out_specs=pl.BlockSpec((c, GSZ), lambda i: (0, i)),
            out_shape=jax.ShapeDtypeStruct((c, tot_h), _F32),
        )(xw_h, af_h, mt_h, w2, b2[:, None])

    th = tot // 4
    halves = [xw[:, i * th:(i + 1) * th] for i in range(4)]
    tc1s = [tc1(xh, th) for xh in halves]
    mts = [_sc_gather_max(bft_h, idx_h, th, c2).reshape(th, c2 + 1)
           for (_, bft_h, idx_h) in tc1s]
    outs = [tc2(xh, af_h, mt_h, th)
            for xh, (af_h, _, _), mt_h in zip(halves, tc1s, mts)]
    out = jnp.concatenate(outs, axis=1)

    o = out.reshape(c, b, nwh, nww, WS, WS)
    o = jnp.transpose(o, (1, 0, 2, 4, 3, 5)).reshape(b, c, h, w)
    return o


# speedup vs baseline: 1.1384x; 1.1384x over previous
import functools

import jax
import jax.numpy as jnp
from jax import lax
from jax.experimental import pallas as pl
from jax.experimental.pallas import tpu as pltpu
from jax.experimental.pallas import tpu_sc as plsc

WS = 8
KNN = 9
EPS_BN = 1e-5
NPW = WS * WS
GW = 8
GSZ = GW * NPW

_F32 = jnp.float32
_I32 = jnp.int32
_HI = lax.Precision.HIGHEST
_RBN = float(1.0 / (1.0 + EPS_BN) ** 0.5)


def _dot(a, b, dims, precision=_HI):
    return lax.dot_general(a, b, (dims, ((), ())),
                           preferred_element_type=_F32, precision=precision)


def _tc1_body(xw_ref, w1_ref, b1_ref, g1_ref, be1_ref, wa_ref, wb_ref,
              bg_ref, af_ref, bft_ref, idx_ref):
    xb = xw_ref[...]

    y = _dot(w1_ref[...], xb, ((1,), (0,)), precision=None)
    y = (y + b1_ref[...]) * _RBN * g1_ref[...] + be1_ref[...]

    ss = jnp.sum(y * y, axis=0, keepdims=True)
    inv = 1.0 / jnp.maximum(jnp.sqrt(ss), 1e-12)
    xn = y * inv
    sq = jnp.sum(xn * xn, axis=0, keepdims=True)

    af = _dot(wa_ref[...], y, ((1,), (0,)), precision=None) + bg_ref[...]
    bf = _dot(wb_ref[...], y, ((1,), (0,)))
    af_ref[...] = af
    bfp = jnp.concatenate([bf, jnp.zeros((1, GSZ), _F32)], axis=0)
    bft_ref[...] = jnp.transpose(bfp)

    dts = []
    for g in range(GW):
        sl = slice(g * NPW, (g + 1) * NPW)
        p = xn[:, sl]
        gm = _dot(p, p, ((0,), (0,)), precision=None)
        sqg = sq[:, sl]
        dts.append((sqg + (-2.0 * gm)) + jnp.transpose(sqg))
    dwt = jnp.concatenate(dts, axis=1)

    rowid = lax.broadcasted_iota(_I32, (NPW, GSZ), 0)
    firsts = []
    for _ in range(KNN):
        cmin = jnp.min(dwt, axis=0, keepdims=True)
        first = jnp.min(jnp.where(dwt == cmin, rowid, NPW),
                        axis=0, keepdims=True)
        firsts.append(first)
        onehot = rowid == first
        dwt = jnp.where(onehot, jnp.inf, dwt)
    firsts += [firsts[0]] * (16 - KNN)
    idx_ref[...] = jnp.concatenate(firsts, axis=0)


def _tc2_body(xw_ref, af_ref, mt_ref, w2_ref, b2_ref, out_ref):
    c2 = af_ref.shape[0]
    m = jnp.transpose(mt_ref[:, :c2])
    e = jnp.maximum(af_ref[...] + m, 0.0)
    out = _dot(w2_ref[...], e, ((1,), (0,)), precision=None) \
        + b2_ref[...] + xw_ref[...]
    out_ref[...] = out


def _sc_gather_max(bft, idx16, tot, c2):
    info = plsc.get_sparse_core_info()
    nw = info.num_cores * info.num_subcores
    wins_per_tile = (tot // NPW) // nw
    st = c2 + 1
    blk = NPW * st
    mesh = plsc.VectorSubcoreMesh(core_axis_name="c", subcore_axis_name="s")

    @functools.partial(
        pl.kernel, mesh=mesh,
        out_type=jax.ShapeDtypeStruct((tot * st,), _F32),
        compiler_params=pltpu.CompilerParams(use_tc_tiling_on_sc=False,
                                             needs_layout_passes=False),
        scratch_types=[
            pltpu.VMEM((2, blk), _F32),
            pltpu.VMEM((2, 16, NPW), _I32),
            pltpu.VMEM((2, blk), _F32),
            pltpu.SemaphoreType.DMA((2,)),
            pltpu.SemaphoreType.DMA((2,)),
        ],
    )
    def sck(bft_hbm, idx_hbm, m_hbm, bft_v, idx_v, m_v, rsem, wsem):
        wid = lax.axis_index("s") * info.num_cores + lax.axis_index("c")

        def start_read(w, sl):
            widx = wid * wins_per_tile + w
            base = widx * NPW
            h1 = pltpu.async_copy(bft_hbm.at[pl.ds(base * st, blk)],
                                  bft_v.at[sl], rsem.at[sl])
            h2 = pltpu.async_copy(idx_hbm.at[widx], idx_v.at[sl],
                                  rsem.at[sl])
            return h1, h2

        hs = {0: start_read(0, 0)}
        ws = {}
        for w in range(wins_per_tile):
            sl = w % 2
            if w + 1 < wins_per_tile:
                hs[w + 1] = start_read(w + 1, (w + 1) % 2)
            for h in hs.pop(w):
                h.wait()
            if w >= 2:
                ws.pop(w - 2).wait()
            for ng in range(NPW // 16):
                jvecs = [idx_v[sl, k, pl.ds(ng * 16, 16)] * st
                         for k in range(KNN)]
                nvec = (lax.broadcasted_iota(_I32, (16,), 0) + ng * 16) * st

                @plsc.parallel_loop(0, c2, unroll=4)
                def chan(c):
                    g = [plsc.load_gather(bft_v.at[sl], [jvecs[k] + c])
                         for k in range(KNN)]
                    while len(g) > 1:
                        g = [jnp.maximum(g[i], g[i + 1])
                             for i in range(0, len(g) - 1, 2)] \
                            + ([g[-1]] if len(g) % 2 else [])
                    plsc.store_scatter(m_v.at[sl], [nvec + c], g[0])
            base = (wid * wins_per_tile + w) * NPW
            ws[w] = pltpu.async_copy(m_v.at[sl],
                                     m_hbm.at[pl.ds(base * st, blk)],
                                     wsem.at[sl])
        for h in ws.values():
            h.wait()

    idx3 = idx16.reshape(16, tot // NPW, NPW).transpose(1, 0, 2)
    return sck(bft.reshape(-1), idx3)


def kernel(x, fc1_w, fc1_b, bn1_g, bn1_b, gc_w, gc_b, gc_bn_g, gc_bn_b,
           fc2_w, fc2_b, bn2_g, bn2_b):
    b, c, h, w = x.shape
    nwh, nww = h // WS, w // WS
    tot = b * nwh * nww * NPW
    c2 = 2 * c

    r = 1.0 / jnp.sqrt(jnp.float32(1.0 + EPS_BN))
    sg = gc_bn_g * r
    wg = gc_w * sg[:, None]
    bgv = gc_b * sg + gc_bn_b
    wa = wg[:, :c] - wg[:, c:]
    wb = wg[:, c:]
    s2 = bn2_g * r
    w2 = fc2_w * s2[:, None]
    b2 = fc2_b * s2 + bn2_b

    xw = x.reshape(b, c, nwh, WS, nww, WS)
    xw = jnp.transpose(xw, (1, 0, 2, 4, 3, 5)).reshape(c, tot)

    def tc1(xw_h, tot_h):
        return pl.pallas_call(
            _tc1_body,
            grid=(tot_h // GSZ,),
            in_specs=[
                pl.BlockSpec((c, GSZ), lambda i: (0, i)),
                pl.BlockSpec((c, c), lambda i: (0, 0)),
                pl.BlockSpec((c, 1), lambda i: (0, 0)),
                pl.BlockSpec((c, 1), lambda i: (0, 0)),
                pl.BlockSpec((c, 1), lambda i: (0, 0)),
                pl.BlockSpec((c2, c), lambda i: (0, 0)),
                pl.BlockSpec((c2, c), lambda i: (0, 0)),
                pl.BlockSpec((c2, 1), lambda i: (0, 0)),
            ],
            out_specs=[
                pl.BlockSpec((c2, GSZ), lambda i: (0, i)),
                pl.BlockSpec((GSZ, c2 + 1), lambda i: (i, 0)),
                pl.BlockSpec((16, GSZ), lambda i: (0, i)),
            ],
            out_shape=[
                jax.ShapeDtypeStruct((c2, tot_h), _F32),
                jax.ShapeDtypeStruct((tot_h, c2 + 1), _F32),
                jax.ShapeDtypeStruct((16, tot_h), _I32),
            ],
        )(xw_h, fc1_w, fc1_b[:, None], bn1_g[:, None], bn1_b[:, None],
          wa, wb, bgv[:, None])

    def tc2(xw_h, af_h, mt_h, tot_h):
        return pl.pallas_call(
            _tc2_body,
            grid=(tot_h // GSZ,),
            in_specs=[
                pl.BlockSpec((c, GSZ), lambda i: (0, i)),
                pl.BlockSpec((c2, GSZ), lambda i: (0, i)),
                pl.BlockSpec((GSZ, c2 + 1), lambda i: (i, 0)),
                pl.BlockSpec((c, c2), lambda i: (0, 0)),
                pl.BlockSpec((c, 1), lambda i: (0, 0)),
            ],
            out_specs=pl.BlockSpec((c, GSZ), lambda i: (0, i)),
            out_shape=jax.ShapeDtypeStruct((c, tot_h), _F32),
        )(xw_h, af_h, mt_h, w2, b2[:, None])

    th = tot // 2
    halves = [xw[:, :th], xw[:, th:]]
    tc1s = [tc1(xh, th) for xh in halves]
    mts = [_sc_gather_max(bft_h, idx_h, th, c2).reshape(th, c2 + 1)
           for (_, bft_h, idx_h) in tc1s]
    outs = [tc2(xh, af_h, mt_h, th)
            for xh, (af_h, _, _), mt_h in zip(halves, tc1s, mts)]
    out = jnp.concatenate(outs, axis=1)

    o = out.reshape(c, b, nwh, nww, WS, WS)
    o = jnp.transpose(o, (1, 0, 2, 4, 3, 5)).reshape(b, c, h, w)
    return o
